# Initial kernel scaffold; baseline (speedup 1.0000x reference)
import jax
import jax.numpy as jnp
from jax.experimental import pallas as pl


def kernel(hidden_states, gate_w, w1_A, w1_B, w2_A, w2_B, w3_A, w3_B):
    raise NotImplementedError("write your pallas kernel here")


# speedup vs baseline: 2.7981x; 2.7981x over previous
import functools
import jax
import jax.numpy as jnp
from jax.experimental import pallas as pl
from jax.experimental.pallas import tpu as pltpu

_B, _S, _H = 1, 2048, 2048
_FFN = 8192
_R = 16
_E = 8
_TOPK = 2
_T = _B * _S

_BT = 256
_FB = 2048


def _dotT(a, b):
    return jax.lax.dot_general(a, b, (((1,), (1,)), ((), ())),
                               preferred_element_type=jnp.float32)


def _dot(a, b):
    return jax.lax.dot_general(a, b, (((1,), (0,)), ((), ())),
                               preferred_element_type=jnp.float32)


def _moe_body(hs_ref, gate_ref, w1a_ref, w1b_ref, w2a_ref, w2b_ref,
              w3a_ref, w3b_ref, out_ref, rw_ref):
    hs = hs_ref[...]

    logits = _dotT(hs, gate_ref[...])
    m = jnp.max(logits, axis=1, keepdims=True)
    p = jnp.exp(logits - m)
    p = p / jnp.sum(p, axis=1, keepdims=True)
    lane = jax.lax.broadcasted_iota(jnp.int32, (_BT, _E), 1)
    i1 = jnp.argmax(p, axis=1).reshape(_BT, 1)
    w1 = jnp.max(p, axis=1, keepdims=True)
    p2 = jnp.where(lane == i1, -1.0, p)
    i2 = jnp.argmax(p2, axis=1).reshape(_BT, 1)
    w2 = jnp.max(p2, axis=1, keepdims=True)
    s = w1 + w2
    w1n = w1 / s
    w2n = w2 / s
    combine = (jnp.where(lane == i1, w1n, 0.0)
               + jnp.where(lane == i2, w2n, 0.0))
    rw_ref[...] = jnp.concatenate([w1n, w2n], axis=1)

    u1 = _dotT(hs, w1a_ref[...])
    u3 = _dotT(hs, w3a_ref[...])

    z_parts = []
    for e in range(_E):
        u1e = u1[:, e * _R:(e + 1) * _R]
        u3e = u3[:, e * _R:(e + 1) * _R]
        acc = jnp.zeros((_BT, _R), dtype=jnp.float32)
        for f in range(_FFN // _FB):
            w1b = w1b_ref[e, :, f * _FB:(f + 1) * _FB]
            w3b = w3b_ref[e, :, f * _FB:(f + 1) * _FB]
            w2a = w2a_ref[e, :, f * _FB:(f + 1) * _FB]
            a1 = _dot(u1e, w1b)
            a3 = _dot(u3e, w3b)
            inter = jnp.where(a1 >= 0.0, a1, 0.01 * a1) * a3
            acc = acc + _dotT(inter, w2a)
        z_parts.append(acc * combine[:, e:e + 1])
    z = jnp.concatenate(z_parts, axis=1)

    out_ref[...] = _dot(z, w2b_ref[...])


@jax.jit
def kernel(hidden_states, gate_w, w1_A, w1_B, w2_A, w2_B, w3_A, w3_B):
    hs = hidden_states.reshape(_T, _H)
    w1a = w1_A.reshape(_E * _R, _H)
    w3a = w3_A.reshape(_E * _R, _H)
    w2b = w2_B.transpose(0, 2, 1).reshape(_E * _R, _H)
    w1b = w1_B.transpose(0, 2, 1)
    w3b = w3_B.transpose(0, 2, 1)

    grid = (_T // _BT,)
    full = lambda shape: pl.BlockSpec(shape, lambda t: (0,) * len(shape))
    out, rw = pl.pallas_call(
        _moe_body,
        grid=grid,
        in_specs=[
            pl.BlockSpec((_BT, _H), lambda t: (t, 0)),
            full((_E, _H)),
            full((_E * _R, _H)),
            full((_E, _R, _FFN)),
            full((_E, _R, _FFN)),
            full((_E * _R, _H)),
            full((_E * _R, _H)),
            full((_E, _R, _FFN)),
        ],
        out_specs=[
            pl.BlockSpec((_BT, _H), lambda t: (t, 0)),
            pl.BlockSpec((_BT, _TOPK), lambda t: (t, 0)),
        ],
        out_shape=[
            jax.ShapeDtypeStruct((_T, _H), jnp.float32),
            jax.ShapeDtypeStruct((_T, _TOPK), jnp.float32),
        ],
        compiler_params=pltpu.CompilerParams(
            dimension_semantics=("arbitrary",),
        ),
    )(hs, gate_w, w1a, w1b, w2_A, w2b, w3a, w3b)
    return out.reshape(_B, _S, _H), rw
